# baseline (speedup 1.0000x reference)
import functools

import jax
import jax.numpy as jnp
from jax import lax
from jax.experimental import pallas as pl
from jax.experimental.pallas import tpu as pltpu
from jax.experimental.pallas import tpu_sc as plsc

EPS = 1e-15
MAX_LOGPROB = 50.0
TOL = 0.001

N_NODES = 10000
N_EDGES = 160000
D_FEAT = 256

NC, NS = 2, 16
NW = NC * NS
E_PAD = 163840
EW = E_PAD // NW
C = 128
NCH = EW // C


def _sc_dots_body(z_hbm, i0_hbm, i1_hbm, out_hbm, i0_v, i1_v, r0, r1, dv, sem0, sem1):
    wid = lax.axis_index("s") * NC + lax.axis_index("c")
    base = wid * EW
    pltpu.sync_copy(i0_hbm.at[pl.ds(base, EW)], i0_v)
    pltpu.sync_copy(i1_hbm.at[pl.ds(base, EW)], i1_v)

    def chunk(c, carry):
        coff = c * C
        cp0 = pltpu.async_copy(z_hbm.at[i0_v.at[pl.ds(coff, C)]], r0, sem0)
        cp1 = pltpu.async_copy(z_hbm.at[i1_v.at[pl.ds(coff, C)]], r1, sem1)
        cp0.wait()
        cp1.wait()

        lane = lax.iota(jnp.int32, 16)

        def group(g, carry2):
            gb = g * 16
            eidx = gb + lane

            def feat(k, acc):
                kvec = jnp.full((16,), 0, jnp.int32) + k
                a = plsc.load_gather(r0, [eidx, kvec])
                b = plsc.load_gather(r1, [eidx, kvec])
                return acc + a * b

            acc = lax.fori_loop(0, D_FEAT, feat, jnp.zeros((16,), jnp.float32),
                                unroll=8)
            dv[pl.ds(pl.multiple_of(coff + gb, 16), 16)] = acc
            return carry2

        lax.fori_loop(0, C // 16, group, 0)
        return carry

    lax.fori_loop(0, NCH, chunk, 0)
    pltpu.sync_copy(dv, out_hbm.at[pl.ds(base, EW)])


@functools.cache
def _get_sc_dots():
    mesh = plsc.VectorSubcoreMesh(core_axis_name="c", subcore_axis_name="s")
    return pl.kernel(
        _sc_dots_body,
        out_type=jax.ShapeDtypeStruct((E_PAD,), jnp.float32),
        mesh=mesh,
        scratch_types=[
            pltpu.VMEM((EW,), jnp.int32),
            pltpu.VMEM((EW,), jnp.int32),
            pltpu.VMEM((C, D_FEAT), jnp.float32),
            pltpu.VMEM((C, D_FEAT), jnp.float32),
            pltpu.VMEM((EW,), jnp.float32),
            pltpu.SemaphoreType.DMA,
            pltpu.SemaphoreType.DMA,
        ],
        compiler_params=pltpu.CompilerParams(
            use_tc_tiling_on_sc=False, needs_layout_passes=False
        ),
    )


def _lgamma(x):
    shift = x * (x + 1.0) * (x + 2.0) * (x + 3.0) * (x + 4.0) * (x + 5.0) \
        * (x + 6.0) * (x + 7.0)
    y = x + 8.0
    yi = 1.0 / y
    y2 = yi * yi
    series = yi * (0.083333333333 + y2 * (-0.002777777778 + y2 * 0.000793650794))
    return (y - 0.5) * jnp.log(y) - y + 0.91893853320467 + series - jnp.log(shift)


def _tc_body(d_ref, i0_ref, i1_ref, x_ref, lp_ref, lg_ref, ln_ref, o_ref):
    e_prec = jnp.exp(lp_ref[0, 0])
    e_gam = jnp.exp(lg_ref[0, 0])
    e_n = jnp.exp(ln_ref[0, 0])
    dfl = jnp.abs(i1_ref[...] - i0_ref[...]).astype(jnp.float32) + 1.0
    diff = jnp.exp(-e_gam * jnp.log(dfl))
    p = 1.0 / (1.0 + jnp.exp(-d_ref[...]))
    alpha = diff * e_prec + p * e_n + EPS
    beta = (1.0 - diff) * e_prec + (1.0 - p) * e_n + EPS
    x = jnp.clip(x_ref[...], TOL, 1.0 - TOL)
    log_prob = (
        (alpha - 1.0) * jnp.log(x)
        + (beta - 1.0) * jnp.log(1.0 - x)
        - (_lgamma(alpha) + _lgamma(beta) - _lgamma(alpha + beta))
    )
    o_ref[...] = jnp.minimum(-log_prob, MAX_LOGPROB)


_ROWS = N_EDGES // 128

_tc_call = pl.pallas_call(
    _tc_body,
    out_shape=jax.ShapeDtypeStruct((_ROWS, 128), jnp.float32),
)


def kernel(z, edge_index, edge_attr, logprecision, loggamma, logN):
    idx0 = edge_index[0]
    idx1 = edge_index[1]
    pad = E_PAD - N_EDGES
    zpad = jnp.zeros((pad,), jnp.int32)
    i0p = jnp.concatenate([idx0, zpad])
    i1p = jnp.concatenate([idx1, zpad])
    dots = _get_sc_dots()(z, i0p, i1p)[:N_EDGES]
    out = _tc_call(
        dots.reshape(_ROWS, 128),
        idx0.reshape(_ROWS, 128),
        idx1.reshape(_ROWS, 128),
        edge_attr.reshape(_ROWS, 128),
        logprecision.reshape(1, 1),
        loggamma.reshape(1, 1),
        logN.reshape(1, 1),
    )
    return out.reshape(N_EDGES)


# speedup vs baseline: 1.3717x; 1.3717x over previous
import functools

import jax
import jax.numpy as jnp
from jax import lax
from jax.experimental import pallas as pl
from jax.experimental.pallas import tpu as pltpu
from jax.experimental.pallas import tpu_sc as plsc

EPS = 1e-15
MAX_LOGPROB = 50.0
TOL = 0.001

N_NODES = 10000
N_EDGES = 160000
D_FEAT = 256

NC, NS = 2, 16
NW = NC * NS
E_PAD = 163840
EW = E_PAD // NW
C = 64
NCH = EW // C


def _sc_dots_body(z_hbm, i0_hbm, i1_hbm, out_hbm, i0_v, i1_v,
                  r0a, r1a, r0b, r1b, dv, sema, semb):
    wid = lax.axis_index("s") * NC + lax.axis_index("c")
    base = wid * EW
    pltpu.sync_copy(i0_hbm.at[pl.ds(base, EW)], i0_v)
    pltpu.sync_copy(i1_hbm.at[pl.ds(base, EW)], i1_v)

    lane = lax.iota(jnp.int32, 16)

    def start(c, r0, r1, sem):
        coff = c * C
        pltpu.async_copy(z_hbm.at[i0_v.at[pl.ds(coff, C)]], r0, sem)
        pltpu.async_copy(z_hbm.at[i1_v.at[pl.ds(coff, C)]], r1, sem)

    def drain(r0, r1, sem):
        pltpu.make_async_copy(z_hbm.at[pl.ds(0, C)], r0, sem).wait()
        pltpu.make_async_copy(z_hbm.at[pl.ds(0, C)], r1, sem).wait()

    def compute(c, r0, r1):
        coff = c * C

        def group(g, carry2):
            gb = g * 16
            eidx = gb + lane

            def feat(k, acc):
                kvec = jnp.full((16,), 0, jnp.int32) + k
                a = plsc.load_gather(r0, [eidx, kvec])
                b = plsc.load_gather(r1, [eidx, kvec])
                return acc + a * b

            acc = lax.fori_loop(0, D_FEAT, feat, jnp.zeros((16,), jnp.float32),
                                unroll=8)
            dv[pl.ds(pl.multiple_of(coff + gb, 16), 16)] = acc
            return carry2

        lax.fori_loop(0, C // 16, group, 0)

    start(0, r0a, r1a, sema)

    def pair(c2, carry):
        c = c2 * 2
        start(c + 1, r0b, r1b, semb)
        drain(r0a, r1a, sema)
        compute(c, r0a, r1a)

        @pl.when(c2 + 1 < NCH // 2)
        def _():
            start(c + 2, r0a, r1a, sema)

        drain(r0b, r1b, semb)
        compute(c + 1, r0b, r1b)
        return carry

    lax.fori_loop(0, NCH // 2, pair, 0)
    pltpu.sync_copy(dv, out_hbm.at[pl.ds(base, EW)])


@functools.cache
def _get_sc_dots():
    mesh = plsc.VectorSubcoreMesh(core_axis_name="c", subcore_axis_name="s")
    return pl.kernel(
        _sc_dots_body,
        out_type=jax.ShapeDtypeStruct((E_PAD,), jnp.float32),
        mesh=mesh,
        scratch_types=[
            pltpu.VMEM((EW,), jnp.int32),
            pltpu.VMEM((EW,), jnp.int32),
            pltpu.VMEM((C, D_FEAT), jnp.float32),
            pltpu.VMEM((C, D_FEAT), jnp.float32),
            pltpu.VMEM((C, D_FEAT), jnp.float32),
            pltpu.VMEM((C, D_FEAT), jnp.float32),
            pltpu.VMEM((EW,), jnp.float32),
            pltpu.SemaphoreType.DMA,
            pltpu.SemaphoreType.DMA,
        ],
        compiler_params=pltpu.CompilerParams(
            use_tc_tiling_on_sc=False, needs_layout_passes=False
        ),
    )


def _lgamma(x):
    shift = x * (x + 1.0) * (x + 2.0) * (x + 3.0) * (x + 4.0) * (x + 5.0) \
        * (x + 6.0) * (x + 7.0)
    y = x + 8.0
    yi = 1.0 / y
    y2 = yi * yi
    series = yi * (0.083333333333 + y2 * (-0.002777777778 + y2 * 0.000793650794))
    return (y - 0.5) * jnp.log(y) - y + 0.91893853320467 + series - jnp.log(shift)


def _tc_body(d_ref, i0_ref, i1_ref, x_ref, lp_ref, lg_ref, ln_ref, o_ref):
    e_prec = jnp.exp(lp_ref[0, 0])
    e_gam = jnp.exp(lg_ref[0, 0])
    e_n = jnp.exp(ln_ref[0, 0])
    dfl = jnp.abs(i1_ref[...] - i0_ref[...]).astype(jnp.float32) + 1.0
    diff = jnp.exp(-e_gam * jnp.log(dfl))
    p = 1.0 / (1.0 + jnp.exp(-d_ref[...]))
    alpha = diff * e_prec + p * e_n + EPS
    beta = (1.0 - diff) * e_prec + (1.0 - p) * e_n + EPS
    x = jnp.clip(x_ref[...], TOL, 1.0 - TOL)
    log_prob = (
        (alpha - 1.0) * jnp.log(x)
        + (beta - 1.0) * jnp.log(1.0 - x)
        - (_lgamma(alpha) + _lgamma(beta) - _lgamma(alpha + beta))
    )
    o_ref[...] = jnp.minimum(-log_prob, MAX_LOGPROB)


_ROWS = N_EDGES // 128

_tc_call = pl.pallas_call(
    _tc_body,
    out_shape=jax.ShapeDtypeStruct((_ROWS, 128), jnp.float32),
)


def kernel(z, edge_index, edge_attr, logprecision, loggamma, logN):
    idx0 = edge_index[0]
    idx1 = edge_index[1]
    pad = E_PAD - N_EDGES
    zpad = jnp.zeros((pad,), jnp.int32)
    i0p = jnp.concatenate([idx0, zpad])
    i1p = jnp.concatenate([idx1, zpad])
    dots = _get_sc_dots()(z, i0p, i1p)[:N_EDGES]
    out = _tc_call(
        dots.reshape(_ROWS, 128),
        idx0.reshape(_ROWS, 128),
        idx1.reshape(_ROWS, 128),
        edge_attr.reshape(_ROWS, 128),
        logprecision.reshape(1, 1),
        loggamma.reshape(1, 1),
        logN.reshape(1, 1),
    )
    return out.reshape(N_EDGES)


# speedup vs baseline: 2.4491x; 1.7855x over previous
import functools

import jax
import jax.numpy as jnp
from jax import lax
from jax.experimental import pallas as pl
from jax.experimental.pallas import tpu as pltpu
from jax.experimental.pallas import tpu_sc as plsc

EPS = 1e-15
MAX_LOGPROB = 50.0
TOL = 0.001

N_NODES = 10000
N_EDGES = 160000
D_FEAT = 256

NC, NS = 2, 16
NW = NC * NS
E_PAD = 163840
EW = E_PAD // NW
C = 64
NCH = EW // C


DW = D_FEAT // 2

def _sc_dots_body(z_hbm, ic_hbm, out_hbm, zsh, ic_v, rba, rbb, dv, sema, semb):
    wid = lax.axis_index("s") * NC + lax.axis_index("c")
    base = wid * EW

    @pl.when(lax.axis_index("s") == 0)
    def _():
        pltpu.sync_copy(z_hbm, zsh)

    pltpu.sync_copy(ic_hbm.at[pl.ds(base * 2, 2 * EW)], ic_v)
    plsc.subcore_barrier()

    lane = lax.iota(jnp.int32, 16)
    mask_hi = jnp.full((16,), -65536, jnp.int32)

    def start(c, rb, sem):
        pltpu.async_copy(zsh.at[ic_v.at[pl.ds(c * 2 * C, 2 * C)]], rb, sem)

    def drain(rb, sem):
        pltpu.make_async_copy(z_hbm.at[pl.ds(0, 2 * C)], rb, sem).wait()

    def compute(c, rb):
        coff = c * C

        def group(g, carry2):
            gb = g * 16
            eidx = gb + lane

            def feat(k, accs):
                acc0, acc1 = accs
                kvec = jnp.full((16,), 0, jnp.int32) + k
                a = plsc.load_gather(rb, [eidx, kvec])
                b = plsc.load_gather(rb, [C + eidx, kvec])
                a0 = plsc.bitcast(lax.shift_left(a, 16), jnp.float32)
                b0 = plsc.bitcast(lax.shift_left(b, 16), jnp.float32)
                a1 = plsc.bitcast(lax.bitwise_and(a, mask_hi), jnp.float32)
                b1 = plsc.bitcast(lax.bitwise_and(b, mask_hi), jnp.float32)
                return acc0 + a0 * b0, acc1 + a1 * b1

            zero = jnp.zeros((16,), jnp.float32)
            acc0, acc1 = lax.fori_loop(0, DW, feat, (zero, zero), unroll=8)
            dv[pl.ds(pl.multiple_of(coff + gb, 16), 16)] = acc0 + acc1
            return carry2

        lax.fori_loop(0, C // 16, group, 0)

    start(0, rba, sema)

    def pair(c2, carry):
        c = c2 * 2
        start(c + 1, rbb, semb)
        drain(rba, sema)
        compute(c, rba)

        @pl.when(c2 + 1 < NCH // 2)
        def _():
            start(c + 2, rba, sema)

        drain(rbb, semb)
        compute(c + 1, rbb)
        return carry

    lax.fori_loop(0, NCH // 2, pair, 0)
    pltpu.sync_copy(dv, out_hbm.at[pl.ds(base, EW)])


@functools.cache
def _get_sc_dots():
    mesh = plsc.VectorSubcoreMesh(core_axis_name="c", subcore_axis_name="s")
    return pl.kernel(
        _sc_dots_body,
        out_type=jax.ShapeDtypeStruct((E_PAD,), jnp.float32),
        mesh=mesh,
        scratch_types=[
            pltpu.VMEM_SHARED((N_NODES, DW), jnp.int32),
            pltpu.VMEM((2 * EW,), jnp.int32),
            pltpu.VMEM((2 * C, DW), jnp.int32),
            pltpu.VMEM((2 * C, DW), jnp.int32),
            pltpu.VMEM((EW,), jnp.float32),
            pltpu.SemaphoreType.DMA,
            pltpu.SemaphoreType.DMA,
        ],
        compiler_params=pltpu.CompilerParams(
            use_tc_tiling_on_sc=False, needs_layout_passes=False
        ),
    )


def _lgamma(x):
    shift = x * (x + 1.0) * (x + 2.0) * (x + 3.0) * (x + 4.0) * (x + 5.0) \
        * (x + 6.0) * (x + 7.0)
    y = x + 8.0
    yi = 1.0 / y
    y2 = yi * yi
    series = yi * (0.083333333333 + y2 * (-0.002777777778 + y2 * 0.000793650794))
    return (y - 0.5) * jnp.log(y) - y + 0.91893853320467 + series - jnp.log(shift)


def _tc_body(d_ref, i0_ref, i1_ref, x_ref, lp_ref, lg_ref, ln_ref, o_ref):
    e_prec = jnp.exp(lp_ref[0, 0])
    e_gam = jnp.exp(lg_ref[0, 0])
    e_n = jnp.exp(ln_ref[0, 0])
    dfl = jnp.abs(i1_ref[...] - i0_ref[...]).astype(jnp.float32) + 1.0
    diff = jnp.exp(-e_gam * jnp.log(dfl))
    p = 1.0 / (1.0 + jnp.exp(-d_ref[...]))
    alpha = diff * e_prec + p * e_n + EPS
    beta = (1.0 - diff) * e_prec + (1.0 - p) * e_n + EPS
    x = jnp.clip(x_ref[...], TOL, 1.0 - TOL)
    log_prob = (
        (alpha - 1.0) * jnp.log(x)
        + (beta - 1.0) * jnp.log(1.0 - x)
        - (_lgamma(alpha) + _lgamma(beta) - _lgamma(alpha + beta))
    )
    o_ref[...] = jnp.minimum(-log_prob, MAX_LOGPROB)


_ROWS = N_EDGES // 128

_tc_call = pl.pallas_call(
    _tc_body,
    out_shape=jax.ShapeDtypeStruct((_ROWS, 128), jnp.float32),
)


def kernel(z, edge_index, edge_attr, logprecision, loggamma, logN):
    idx0 = edge_index[0]
    idx1 = edge_index[1]
    pad = E_PAD - N_EDGES
    zpad = jnp.zeros((pad,), jnp.int32)
    i0p = jnp.concatenate([idx0, zpad]).reshape(NW, NCH, C)
    i1p = jnp.concatenate([idx1, zpad]).reshape(NW, NCH, C)
    ic = jnp.stack([i0p, i1p], axis=2).reshape(2 * E_PAD)
    z_packed = lax.bitcast_convert_type(
        z.astype(jnp.bfloat16).reshape(N_NODES, DW, 2), jnp.int32)
    dots = _get_sc_dots()(z_packed, ic)[:N_EDGES]
    out = _tc_call(
        dots.reshape(_ROWS, 128),
        idx0.reshape(_ROWS, 128),
        idx1.reshape(_ROWS, 128),
        edge_attr.reshape(_ROWS, 128),
        logprecision.reshape(1, 1),
        loggamma.reshape(1, 1),
        logN.reshape(1, 1),
    )
    return out.reshape(N_EDGES)


# speedup vs baseline: 3.9558x; 1.6152x over previous
import functools

import jax
import jax.numpy as jnp
from jax import lax
from jax.experimental import pallas as pl
from jax.experimental.pallas import tpu as pltpu
from jax.experimental.pallas import tpu_sc as plsc

EPS = 1e-15
MAX_LOGPROB = 50.0
TOL = 0.001

N_NODES = 10000
N_EDGES = 160000
D_FEAT = 256

NC, NS = 2, 16
NW = NC * NS
E_PAD = 163840
ESC = E_PAD // NC
CH = 4096
NCHS = ESC // CH
DW = D_FEAT // 2
WPT = DW // NS
SROW = 64
CROWS = CH // SROW


def _sc_dots_body(zt_hbm, ic_hbm, out_hbm, zt, iba, ibb, pb, ridx, shacc,
                  sema, semb):
    cid = lax.axis_index("c")
    sid = lax.axis_index("s")
    lane = lax.iota(jnp.int32, 16)
    mask_hi = jnp.full((16,), -65536, jnp.int32)
    zeros16 = jnp.zeros((16,), jnp.float32)

    pltpu.sync_copy(zt_hbm.at[sid], zt)

    @pl.when(sid == 0)
    def _():
        def zrow(r, carry):
            for j in range(SROW // 16):
                pb[r, pl.ds(j * 16, 16)] = zeros16
            return carry

        lax.fori_loop(0, CROWS, zrow, 0)

        def zcp(i, carry):
            pltpu.sync_copy(pb, shacc.at[pl.ds(i * CROWS, CROWS)])
            return carry

        lax.fori_loop(0, ESC // CH, zcp, 0)

    plsc.subcore_barrier()

    icbase = cid * (NCHS * 2 * CH)

    def start(ch, ib, sem):
        pltpu.async_copy(ic_hbm.at[pl.ds(icbase + ch * 2 * CH, 2 * CH)],
                         ib, sem)

    def drain(ib, sem):
        pltpu.make_async_copy(ic_hbm.at[pl.ds(0, 2 * CH)], ib, sem).wait()

    def compute(ch, ib):
        rbase = ch * CROWS
        for j in range(CROWS // 16):
            ridx[pl.ds(j * 16, 16)] = rbase + j * 16 + lane

        def group(g, carry):
            goff = g * 16
            n0 = ib[pl.ds(goff, 16)]
            n1 = ib[pl.ds(CH + goff, 16)]
            acc0 = zeros16
            acc1 = zeros16
            for kk in range(WPT):
                kv = jnp.full((16,), kk, jnp.int32)
                a = plsc.load_gather(zt, [n0, kv])
                b = plsc.load_gather(zt, [n1, kv])
                a0 = plsc.bitcast(lax.shift_left(a, 16), jnp.float32)
                b0 = plsc.bitcast(lax.shift_left(b, 16), jnp.float32)
                a1 = plsc.bitcast(lax.bitwise_and(a, mask_hi), jnp.float32)
                b1 = plsc.bitcast(lax.bitwise_and(b, mask_hi), jnp.float32)
                acc0 = acc0 + a0 * b0
                acc1 = acc1 + a1 * b1
            row = lax.shift_right_logical(g, 2)
            col = lax.mul(lax.bitwise_and(g, 3), 16)
            pb[row, pl.ds(pl.multiple_of(col, 16), 16)] = acc0 + acc1
            return carry

        lax.fori_loop(0, CH // 16, group, 0)
        pltpu.sync_copy(pb, shacc.at[ridx], add=True)

    start(0, iba, sema)

    def pair(c2, carry):
        ch = c2 * 2
        start(ch + 1, ibb, semb)
        drain(iba, sema)
        compute(ch, iba)

        @pl.when(c2 + 1 < NCHS // 2)
        def _():
            start(ch + 2, iba, sema)

        drain(ibb, semb)
        compute(ch + 1, ibb)
        return carry

    lax.fori_loop(0, NCHS // 2, pair, 0)
    plsc.subcore_barrier()

    spt = (ESC // SROW) // NS
    pltpu.sync_copy(shacc.at[pl.ds(sid * spt, spt)],
                    out_hbm.at[pl.ds(cid * (ESC // SROW) + sid * spt, spt)])


@functools.cache
def _get_sc_dots():
    mesh = plsc.VectorSubcoreMesh(core_axis_name="c", subcore_axis_name="s")
    return pl.kernel(
        _sc_dots_body,
        out_type=jax.ShapeDtypeStruct((E_PAD // SROW, SROW), jnp.float32),
        mesh=mesh,
        scratch_types=[
            pltpu.VMEM((N_NODES, WPT), jnp.int32),
            pltpu.VMEM((2 * CH,), jnp.int32),
            pltpu.VMEM((2 * CH,), jnp.int32),
            pltpu.VMEM((CROWS, SROW), jnp.float32),
            pltpu.VMEM((CROWS,), jnp.int32),
            pltpu.VMEM_SHARED((ESC // SROW, SROW), jnp.float32),
            pltpu.SemaphoreType.DMA,
            pltpu.SemaphoreType.DMA,
        ],
        compiler_params=pltpu.CompilerParams(
            use_tc_tiling_on_sc=False, needs_layout_passes=False
        ),
    )


def _lgamma(x):
    shift = x * (x + 1.0) * (x + 2.0) * (x + 3.0) * (x + 4.0) * (x + 5.0) \
        * (x + 6.0) * (x + 7.0)
    y = x + 8.0
    yi = 1.0 / y
    y2 = yi * yi
    series = yi * (0.083333333333 + y2 * (-0.002777777778 + y2 * 0.000793650794))
    return (y - 0.5) * jnp.log(y) - y + 0.91893853320467 + series - jnp.log(shift)


def _tc_body(d_ref, i0_ref, i1_ref, x_ref, lp_ref, lg_ref, ln_ref, o_ref):
    e_prec = jnp.exp(lp_ref[0, 0])
    e_gam = jnp.exp(lg_ref[0, 0])
    e_n = jnp.exp(ln_ref[0, 0])
    dfl = jnp.abs(i1_ref[...] - i0_ref[...]).astype(jnp.float32) + 1.0
    diff = jnp.exp(-e_gam * jnp.log(dfl))
    p = 1.0 / (1.0 + jnp.exp(-d_ref[...]))
    alpha = diff * e_prec + p * e_n + EPS
    beta = (1.0 - diff) * e_prec + (1.0 - p) * e_n + EPS
    x = jnp.clip(x_ref[...], TOL, 1.0 - TOL)
    log_prob = (
        (alpha - 1.0) * jnp.log(x)
        + (beta - 1.0) * jnp.log(1.0 - x)
        - (_lgamma(alpha) + _lgamma(beta) - _lgamma(alpha + beta))
    )
    o_ref[...] = jnp.minimum(-log_prob, MAX_LOGPROB)


_ROWS = N_EDGES // 128

_tc_call = pl.pallas_call(
    _tc_body,
    out_shape=jax.ShapeDtypeStruct((_ROWS, 128), jnp.float32),
)


def kernel(z, edge_index, edge_attr, logprecision, loggamma, logN):
    idx0 = edge_index[0]
    idx1 = edge_index[1]
    pad = E_PAD - N_EDGES
    zpad = jnp.zeros((pad,), jnp.int32)
    i0p = jnp.concatenate([idx0, zpad]).reshape(NC, NCHS, CH)
    i1p = jnp.concatenate([idx1, zpad]).reshape(NC, NCHS, CH)
    ic = jnp.stack([i0p, i1p], axis=2).reshape(2 * E_PAD)
    z_packed = lax.bitcast_convert_type(
        z.astype(jnp.bfloat16).reshape(N_NODES, DW, 2), jnp.int32)
    z_tiles = z_packed.reshape(N_NODES, NS, WPT).transpose(1, 0, 2)
    dots = _get_sc_dots()(z_tiles, ic).reshape(E_PAD)[:N_EDGES]
    out = _tc_call(
        dots.reshape(_ROWS, 128),
        idx0.reshape(_ROWS, 128),
        idx1.reshape(_ROWS, 128),
        edge_attr.reshape(_ROWS, 128),
        logprecision.reshape(1, 1),
        loggamma.reshape(1, 1),
        logN.reshape(1, 1),
    )
    return out.reshape(N_EDGES)


# speedup vs baseline: 4.1289x; 1.0437x over previous
import functools

import jax
import jax.numpy as jnp
from jax import lax
from jax.experimental import pallas as pl
from jax.experimental.pallas import tpu as pltpu
from jax.experimental.pallas import tpu_sc as plsc

EPS = 1e-15
MAX_LOGPROB = 50.0
TOL = 0.001

N_NODES = 10000
N_EDGES = 160000
D_FEAT = 256

NC, NS = 2, 16
NW = NC * NS
E_PAD = 163840
ESC = E_PAD // NC
CH = 4096
NCHS = ESC // CH
DW = D_FEAT // 2
WPT = DW // NS
SROW = 64
CROWS = CH // SROW


def _sc_dots_body(zt_hbm, ic_hbm, out_hbm, zt, iba, ibb, pba, pbb,
                  sema, semb, semp):
    cid = lax.axis_index("c")
    sid = lax.axis_index("s")
    mask_hi = jnp.full((16,), -65536, jnp.int32)
    zeros16 = jnp.zeros((16,), jnp.float32)

    pltpu.sync_copy(zt_hbm.at[sid], zt)

    icbase = cid * (NCHS * 2 * CH)
    outbase = cid * ESC

    def start(ch, ib, sem):
        pltpu.async_copy(ic_hbm.at[pl.ds(icbase + ch * 2 * CH, 2 * CH)],
                         ib, sem)

    def drain(ib, sem):
        pltpu.make_async_copy(ic_hbm.at[pl.ds(0, 2 * CH)], ib, sem).wait()

    def drain_p(pb):
        pltpu.make_async_copy(pb, out_hbm.at[sid, pl.ds(0, CH)], semp).wait()

    def compute(ch, ib, pb):
        def group(g, carry):
            goff = g * 16
            n0 = ib[pl.ds(goff, 16)]
            n1 = ib[pl.ds(CH + goff, 16)]
            acc0 = zeros16
            acc1 = zeros16
            for kk in range(WPT):
                kv = jnp.full((16,), kk, jnp.int32)
                a = plsc.load_gather(zt, [n0, kv])
                b = plsc.load_gather(zt, [n1, kv])
                a0 = plsc.bitcast(lax.shift_left(a, 16), jnp.float32)
                b0 = plsc.bitcast(lax.shift_left(b, 16), jnp.float32)
                a1 = plsc.bitcast(lax.bitwise_and(a, mask_hi), jnp.float32)
                b1 = plsc.bitcast(lax.bitwise_and(b, mask_hi), jnp.float32)
                acc0 = acc0 + a0 * b0
                acc1 = acc1 + a1 * b1
            pb[pl.ds(pl.multiple_of(goff, 16), 16)] = acc0 + acc1
            return carry

        lax.fori_loop(0, CH // 16, group, 0)
        pltpu.async_copy(pb, out_hbm.at[sid, pl.ds(outbase + ch * CH, CH)],
                         semp)

    start(0, iba, sema)

    def pair(c2, carry):
        ch = c2 * 2
        start(ch + 1, ibb, semb)
        drain(iba, sema)

        @pl.when(c2 > 0)
        def _():
            drain_p(pba)

        compute(ch, iba, pba)

        @pl.when(c2 + 1 < NCHS // 2)
        def _():
            start(ch + 2, iba, sema)

        drain(ibb, semb)

        @pl.when(c2 > 0)
        def _():
            drain_p(pbb)

        compute(ch + 1, ibb, pbb)
        return carry

    lax.fori_loop(0, NCHS // 2, pair, 0)
    drain_p(pba)
    drain_p(pbb)


@functools.cache
def _get_sc_dots():
    mesh = plsc.VectorSubcoreMesh(core_axis_name="c", subcore_axis_name="s")
    return pl.kernel(
        _sc_dots_body,
        out_type=jax.ShapeDtypeStruct((NS, E_PAD), jnp.float32),
        mesh=mesh,
        scratch_types=[
            pltpu.VMEM((N_NODES, WPT), jnp.int32),
            pltpu.VMEM((2 * CH,), jnp.int32),
            pltpu.VMEM((2 * CH,), jnp.int32),
            pltpu.VMEM((CH,), jnp.float32),
            pltpu.VMEM((CH,), jnp.float32),
            pltpu.SemaphoreType.DMA,
            pltpu.SemaphoreType.DMA,
            pltpu.SemaphoreType.DMA,
        ],
        compiler_params=pltpu.CompilerParams(
            use_tc_tiling_on_sc=False, needs_layout_passes=False
        ),
    )


def _lgamma(x):
    shift = x * (x + 1.0) * (x + 2.0) * (x + 3.0) * (x + 4.0) * (x + 5.0) \
        * (x + 6.0) * (x + 7.0)
    y = x + 8.0
    yi = 1.0 / y
    y2 = yi * yi
    series = yi * (0.083333333333 + y2 * (-0.002777777778 + y2 * 0.000793650794))
    return (y - 0.5) * jnp.log(y) - y + 0.91893853320467 + series - jnp.log(shift)


def _tc_body(d_ref, i0_ref, i1_ref, x_ref, lp_ref, lg_ref, ln_ref, o_ref):
    e_prec = jnp.exp(lp_ref[0, 0])
    e_gam = jnp.exp(lg_ref[0, 0])
    e_n = jnp.exp(ln_ref[0, 0])
    dfl = jnp.abs(i1_ref[...] - i0_ref[...]).astype(jnp.float32) + 1.0
    diff = jnp.exp(-e_gam * jnp.log(dfl))
    dots = jnp.sum(d_ref[...], axis=0)
    p = 1.0 / (1.0 + jnp.exp(-dots))
    alpha = diff * e_prec + p * e_n + EPS
    beta = (1.0 - diff) * e_prec + (1.0 - p) * e_n + EPS
    x = jnp.clip(x_ref[...], TOL, 1.0 - TOL)
    log_prob = (
        (alpha - 1.0) * jnp.log(x)
        + (beta - 1.0) * jnp.log(1.0 - x)
        - (_lgamma(alpha) + _lgamma(beta) - _lgamma(alpha + beta))
    )
    o_ref[...] = jnp.minimum(-log_prob, MAX_LOGPROB)


_RPAD = E_PAD // 128

_tc_call = pl.pallas_call(
    _tc_body,
    out_shape=jax.ShapeDtypeStruct((_RPAD, 128), jnp.float32),
)


def kernel(z, edge_index, edge_attr, logprecision, loggamma, logN):
    idx0 = edge_index[0]
    idx1 = edge_index[1]
    pad = E_PAD - N_EDGES
    zpad = jnp.zeros((pad,), jnp.int32)
    i0p = jnp.concatenate([idx0, zpad])
    i1p = jnp.concatenate([idx1, zpad])
    ic = jnp.stack([i0p.reshape(NC, NCHS, CH), i1p.reshape(NC, NCHS, CH)],
                   axis=2).reshape(2 * E_PAD)
    xp = jnp.concatenate([edge_attr, jnp.full((pad,), 0.5, jnp.float32)])
    z_packed = lax.bitcast_convert_type(
        z.astype(jnp.bfloat16).reshape(N_NODES, DW, 2), jnp.int32)
    z_tiles = z_packed.reshape(N_NODES, NS, WPT).transpose(1, 0, 2)
    partials = _get_sc_dots()(z_tiles, ic)
    out = _tc_call(
        partials.reshape(NS, _RPAD, 128),
        i0p.reshape(_RPAD, 128),
        i1p.reshape(_RPAD, 128),
        xp.reshape(_RPAD, 128),
        logprecision.reshape(1, 1),
        loggamma.reshape(1, 1),
        logN.reshape(1, 1),
    )
    return out.reshape(E_PAD)[:N_EDGES]


# speedup vs baseline: 4.1444x; 1.0037x over previous
import functools

import jax
import jax.numpy as jnp
from jax import lax
from jax.experimental import pallas as pl
from jax.experimental.pallas import tpu as pltpu
from jax.experimental.pallas import tpu_sc as plsc

EPS = 1e-15
MAX_LOGPROB = 50.0
TOL = 0.001

N_NODES = 10000
N_EDGES = 160000
D_FEAT = 256

NC, NS = 2, 16
NW = NC * NS
E_PAD = 163840
ESC = E_PAD // NC
CH = 4096
NCHS = ESC // CH
DW = D_FEAT // 2
WPT = DW // NS
SROW = 64
CROWS = CH // SROW


def _sc_dots_body(zt_hbm, ic_hbm, out_hbm, zt, iba, ibb, pba, pbb,
                  sema, semb, semp):
    cid = lax.axis_index("c")
    sid = lax.axis_index("s")
    zeros16 = jnp.zeros((16,), jnp.float32)

    pltpu.sync_copy(zt_hbm.at[sid], zt)

    icbase = cid * (NCHS * 2 * CH)
    outbase = cid * ESC

    def start(ch, ib, sem):
        pltpu.async_copy(ic_hbm.at[pl.ds(icbase + ch * 2 * CH, 2 * CH)],
                         ib, sem)

    def drain(ib, sem):
        pltpu.make_async_copy(ic_hbm.at[pl.ds(0, 2 * CH)], ib, sem).wait()

    def drain_p(pb):
        pltpu.make_async_copy(pb, out_hbm.at[sid, pl.ds(0, CH)], semp).wait()

    def compute(ch, ib, pb):
        def group(g, carry):
            goff = g * 16
            n0 = ib[pl.ds(goff, 16)]
            n1 = ib[pl.ds(CH + goff, 16)]
            acc0 = zeros16
            acc1 = zeros16
            for kk in range(WPT):
                kv = jnp.full((16,), kk, jnp.int32)
                a = plsc.load_gather(zt, [n0, kv])
                b = plsc.load_gather(zt, [n1, kv])
                prod = (plsc.bitcast(a, jnp.bfloat16)
                        * plsc.bitcast(b, jnp.bfloat16))
                p0, p1 = plsc.unpack(prod, format=plsc.PackFormat.INTERLEAVED)
                acc0 = acc0 + p0
                acc1 = acc1 + p1
            pb[pl.ds(pl.multiple_of(goff, 16), 16)] = acc0 + acc1
            return carry

        lax.fori_loop(0, CH // 16, group, 0)
        pltpu.async_copy(pb, out_hbm.at[sid, pl.ds(outbase + ch * CH, CH)],
                         semp)

    start(0, iba, sema)

    def pair(c2, carry):
        ch = c2 * 2
        start(ch + 1, ibb, semb)
        drain(iba, sema)

        @pl.when(c2 > 0)
        def _():
            drain_p(pba)

        compute(ch, iba, pba)

        @pl.when(c2 + 1 < NCHS // 2)
        def _():
            start(ch + 2, iba, sema)

        drain(ibb, semb)

        @pl.when(c2 > 0)
        def _():
            drain_p(pbb)

        compute(ch + 1, ibb, pbb)
        return carry

    lax.fori_loop(0, NCHS // 2, pair, 0)
    drain_p(pba)
    drain_p(pbb)


@functools.cache
def _get_sc_dots():
    mesh = plsc.VectorSubcoreMesh(core_axis_name="c", subcore_axis_name="s")
    return pl.kernel(
        _sc_dots_body,
        out_type=jax.ShapeDtypeStruct((NS, E_PAD), jnp.float32),
        mesh=mesh,
        scratch_types=[
            pltpu.VMEM((N_NODES, WPT), jnp.int32),
            pltpu.VMEM((2 * CH,), jnp.int32),
            pltpu.VMEM((2 * CH,), jnp.int32),
            pltpu.VMEM((CH,), jnp.float32),
            pltpu.VMEM((CH,), jnp.float32),
            pltpu.SemaphoreType.DMA,
            pltpu.SemaphoreType.DMA,
            pltpu.SemaphoreType.DMA,
        ],
        compiler_params=pltpu.CompilerParams(
            use_tc_tiling_on_sc=False, needs_layout_passes=False
        ),
    )


def _lgamma(x):
    shift = x * (x + 1.0) * (x + 2.0) * (x + 3.0) * (x + 4.0) * (x + 5.0) \
        * (x + 6.0) * (x + 7.0)
    y = x + 8.0
    yi = 1.0 / y
    y2 = yi * yi
    series = yi * (0.083333333333 + y2 * (-0.002777777778 + y2 * 0.000793650794))
    return (y - 0.5) * jnp.log(y) - y + 0.91893853320467 + series - jnp.log(shift)


def _tc_body(d_ref, i0_ref, i1_ref, x_ref, lp_ref, lg_ref, ln_ref, o_ref):
    e_prec = jnp.exp(lp_ref[0, 0])
    e_gam = jnp.exp(lg_ref[0, 0])
    e_n = jnp.exp(ln_ref[0, 0])
    dfl = jnp.abs(i1_ref[...] - i0_ref[...]).astype(jnp.float32) + 1.0
    diff = jnp.exp(-e_gam * jnp.log(dfl))
    dots = jnp.sum(d_ref[...], axis=0)
    p = 1.0 / (1.0 + jnp.exp(-dots))
    alpha = diff * e_prec + p * e_n + EPS
    beta = (1.0 - diff) * e_prec + (1.0 - p) * e_n + EPS
    x = jnp.clip(x_ref[...], TOL, 1.0 - TOL)
    log_prob = (
        (alpha - 1.0) * jnp.log(x)
        + (beta - 1.0) * jnp.log(1.0 - x)
        - (_lgamma(alpha) + _lgamma(beta) - _lgamma(alpha + beta))
    )
    o_ref[...] = jnp.minimum(-log_prob, MAX_LOGPROB)


_RPAD = E_PAD // 128

_tc_call = pl.pallas_call(
    _tc_body,
    out_shape=jax.ShapeDtypeStruct((_RPAD, 128), jnp.float32),
)


def kernel(z, edge_index, edge_attr, logprecision, loggamma, logN):
    idx0 = edge_index[0]
    idx1 = edge_index[1]
    pad = E_PAD - N_EDGES
    zpad = jnp.zeros((pad,), jnp.int32)
    i0p = jnp.concatenate([idx0, zpad])
    i1p = jnp.concatenate([idx1, zpad])
    ic = jnp.stack([i0p.reshape(NC, NCHS, CH), i1p.reshape(NC, NCHS, CH)],
                   axis=2).reshape(2 * E_PAD)
    xp = jnp.concatenate([edge_attr, jnp.full((pad,), 0.5, jnp.float32)])
    z_packed = lax.bitcast_convert_type(
        z.astype(jnp.bfloat16).reshape(N_NODES, DW, 2), jnp.int32)
    z_tiles = z_packed.reshape(N_NODES, NS, WPT).transpose(1, 0, 2)
    partials = _get_sc_dots()(z_tiles, ic)
    out = _tc_call(
        partials.reshape(NS, _RPAD, 128),
        i0p.reshape(_RPAD, 128),
        i1p.reshape(_RPAD, 128),
        xp.reshape(_RPAD, 128),
        logprecision.reshape(1, 1),
        loggamma.reshape(1, 1),
        logN.reshape(1, 1),
    )
    return out.reshape(E_PAD)[:N_EDGES]
